# baseline (speedup 1.0000x reference)
import functools

import jax
import jax.numpy as jnp
from jax.experimental import pallas as pl
from jax.experimental.pallas import tpu as pltpu
from jax.experimental.pallas import tpu_sc as plsc

EMBED = 512
EXPERTS = 16
TOPK = 4
HEADS = 16
DEPTH = 2
NUM_CLASSES = 1000
POOL = 14
B, P, C, H, W = 16, 16, 3, 56, 56
N = B * P
HD = EMBED // HEADS


def _ln(x, g, b):
    m = jnp.mean(x, axis=-1, keepdims=True)
    v = jnp.mean((x - m) ** 2, axis=-1, keepdims=True)
    return (x - m) / jnp.sqrt(v + 1e-6) * g + b


def _stem_body(x_ref, m_ref, w_ref, b_ref, o_ref):
    xb = x_ref[...]
    nc = xb.shape[0] // (C * H)
    x4 = xb.reshape(nc * C, POOL, 4, W)
    hs = x4[:, :, 0, :] + x4[:, :, 1, :] + x4[:, :, 2, :] + x4[:, :, 3, :]
    hs = hs.reshape(nc * C * POOL, W)
    s = jnp.dot(hs, m_ref[...], preferred_element_type=jnp.float32,
                precision=jax.lax.Precision.HIGHEST)
    s3 = s.reshape(nc, C * POOL, POOL)
    w3 = w_ref[...].reshape(C * POOL, POOL, EMBED)
    feats = b_ref[...]
    for k in range(C * POOL):
        feats = feats + jnp.dot(s3[:, k, :], w3[k],
                                preferred_element_type=jnp.float32)
    o_ref[...] = feats


def _stem(x2, w2, b):
    chunks = 8
    nc = N // chunks
    m = (jnp.arange(W)[:, None] // 4 == jnp.arange(POOL)[None, :])
    m = m.astype(jnp.float32) / 16.0
    return pl.pallas_call(
        _stem_body,
        grid=(chunks,),
        in_specs=[
            pl.BlockSpec((nc * C * H, W), lambda i: (i, 0)),
            pl.BlockSpec((W, POOL), lambda i: (0, 0)),
            pl.BlockSpec((C * POOL * POOL, EMBED), lambda i: (0, 0)),
            pl.BlockSpec((1, EMBED), lambda i: (0, 0)),
        ],
        out_specs=pl.BlockSpec((nc, EMBED), lambda i: (i, 0)),
        out_shape=jax.ShapeDtypeStruct((N, EMBED), jnp.float32),
    )(x2, m, w2, b)


def _attn_body(h_ref, g_ref, bln_ref, wqkv_ref, bqkv_ref, wo_ref, bo_ref,
               g2_ref, b2_ref, wg_ref, o_ref, y_ref, lg_ref):
    h = h_ref[0]
    y = _ln(h, g_ref[...], bln_ref[...])
    qkv = jnp.dot(y, wqkv_ref[...], preferred_element_type=jnp.float32)
    qkv = qkv + bqkv_ref[...]

    qkvT = qkv.T
    q3 = qkvT[:EMBED].reshape(HEADS, HD, P)
    k3 = qkvT[EMBED:2 * EMBED].reshape(HEADS, HD, P)
    v3 = qkvT[2 * EMBED:].reshape(HEADS, HD, P)
    s = jax.lax.dot_general(q3, k3, (((1,), (1,)), ((0,), (0,))),
                            preferred_element_type=jnp.float32)
    s = s / jnp.sqrt(jnp.float32(HD))
    s = s - jnp.max(s, axis=-1, keepdims=True)
    ex = jnp.exp(s)
    att = ex / jnp.sum(ex, axis=-1, keepdims=True)
    o3 = jax.lax.dot_general(v3, att, (((2,), (2,)), ((0,), (0,))),
                             preferred_element_type=jnp.float32)
    wo = wo_ref[...]
    acc = h + bo_ref[...]
    for hh in range(HEADS):
        acc = acc + jax.lax.dot_general(
            o3[hh], wo[HD * hh:HD * (hh + 1), :], (((0,), (0,)), ((), ())),
            preferred_element_type=jnp.float32)
    o_ref[0] = acc
    y2 = _ln(acc, g2_ref[...], b2_ref[...])
    y_ref[0] = y2
    lg_ref[0] = jnp.dot(y2, wg_ref[...], preferred_element_type=jnp.float32)


def _attn(h3, g, bln, wqkv, bqkv, wo, bo, g2, b2, wg):
    return pl.pallas_call(
        _attn_body,
        grid=(B,),
        in_specs=[
            pl.BlockSpec((1, P, EMBED), lambda b: (b, 0, 0)),
            pl.BlockSpec((1, EMBED), lambda b: (0, 0)),
            pl.BlockSpec((1, EMBED), lambda b: (0, 0)),
            pl.BlockSpec((EMBED, 3 * EMBED), lambda b: (0, 0)),
            pl.BlockSpec((1, 3 * EMBED), lambda b: (0, 0)),
            pl.BlockSpec((EMBED, EMBED), lambda b: (0, 0)),
            pl.BlockSpec((1, EMBED), lambda b: (0, 0)),
            pl.BlockSpec((1, EMBED), lambda b: (0, 0)),
            pl.BlockSpec((1, EMBED), lambda b: (0, 0)),
            pl.BlockSpec((EMBED, EXPERTS), lambda b: (0, 0)),
        ],
        out_specs=[
            pl.BlockSpec((1, P, EMBED), lambda b: (b, 0, 0)),
            pl.BlockSpec((1, P, EMBED), lambda b: (b, 0, 0)),
            pl.BlockSpec((1, P, EXPERTS), lambda b: (b, 0, 0)),
        ],
        out_shape=[
            jax.ShapeDtypeStruct((B, P, EMBED), jnp.float32),
            jax.ShapeDtypeStruct((B, P, EMBED), jnp.float32),
            jax.ShapeDtypeStruct((B, P, EXPERTS), jnp.float32),
        ],
    )(h3, g, bln, wqkv, bqkv, wo, bo, g2, b2, wg)


def _comb_sc(logits):
    info = plsc.get_sparse_core_info()
    nw = info.num_cores * info.num_subcores
    rows = N // nw
    mesh = plsc.VectorSubcoreMesh(core_axis_name="c", subcore_axis_name="s")

    @functools.partial(
        pl.kernel, mesh=mesh,
        out_type=jax.ShapeDtypeStruct((N, EXPERTS), jnp.float32),
        scratch_types=[
            pltpu.VMEM((rows, EXPERTS), jnp.float32),
            pltpu.VMEM((rows, EXPERTS), jnp.float32),
        ],
        compiler_params=pltpu.CompilerParams(needs_layout_passes=False),
    )
    def k(l_hbm, out_hbm, l_v, o_v):
        wid = (jax.lax.axis_index("s") * info.num_cores
               + jax.lax.axis_index("c"))
        base = wid * rows
        pltpu.sync_copy(l_hbm.at[pl.ds(base, rows)], l_v)
        iot = jax.lax.iota(jnp.int32, 16)

        def rev(x):
            return jax.lax.rev(x, (0,))

        def bmax(x):
            return plsc.cummax(rev(plsc.cummax(x)))

        def bsum(x):
            return plsc.cumsum(x) + rev(plsc.cumsum(rev(x))) - x

        for r in range(rows):
            l = l_v[r]
            ex = jnp.exp(l - bmax(l))
            p = ex / bsum(ex)
            avail = iot >= 0
            sel = iot < 0
            for _ in range(TOPK):
                pm = jnp.where(avail, p, -1e30)
                mx = bmax(pm)
                cand = pm == mx
                idx = -bmax(jnp.where(cand, -iot, -EXPERTS))
                hit = iot == idx
                sel = sel | hit
                avail = avail & (~hit)
            cw = jnp.where(sel, p, 0.0)
            o_v[r] = cw / bsum(cw)
        pltpu.sync_copy(o_v, out_hbm.at[pl.ds(base, rows)])

    return k(logits)


def _moe_body(h_ref, y_ref, comb_ref, we1_ref, be1_ref, we2_ref,
              be2_ref, o_ref, acc_s):
    e = pl.program_id(0)
    y = y_ref[...]
    t = jnp.dot(y, we1_ref[0], preferred_element_type=jnp.float32)
    h1 = jax.nn.gelu(t + be1_ref[0])
    h2 = jnp.dot(h1, we2_ref[0], preferred_element_type=jnp.float32)
    h2 = h2 + be2_ref[0]
    lane = jax.lax.broadcasted_iota(jnp.int32, (N, EXPERTS), 1)
    ce = jnp.sum(comb_ref[...] * (lane == e).astype(jnp.float32), axis=1,
                 keepdims=True)
    contrib = (ce.astype(jnp.bfloat16).astype(jnp.float32) *
               h2.astype(jnp.bfloat16).astype(jnp.float32))

    @pl.when(e == 0)
    def _first():
        acc_s[...] = contrib

    @pl.when(e > 0)
    def _rest():
        acc_s[...] += contrib

    @pl.when(e == EXPERTS - 1)
    def _final():
        o_ref[...] = h_ref[...] + acc_s[...]


def _moe(h2d, y2d, comb, we1, be1, we2, be2):
    return pl.pallas_call(
        _moe_body,
        grid=(EXPERTS,),
        in_specs=[
            pl.BlockSpec((N, EMBED), lambda e: (0, 0)),
            pl.BlockSpec((N, EMBED), lambda e: (0, 0)),
            pl.BlockSpec((N, EXPERTS), lambda e: (0, 0)),
            pl.BlockSpec((1, EMBED, EMBED), lambda e: (e, 0, 0)),
            pl.BlockSpec((1, 1, EMBED), lambda e: (e, 0, 0)),
            pl.BlockSpec((1, EMBED, EMBED), lambda e: (e, 0, 0)),
            pl.BlockSpec((1, 1, EMBED), lambda e: (e, 0, 0)),
        ],
        out_specs=pl.BlockSpec((N, EMBED), lambda e: (0, 0)),
        out_shape=jax.ShapeDtypeStruct((N, EMBED), jnp.float32),
        scratch_shapes=[
            pltpu.VMEM((N, EMBED), jnp.float32),
        ],
    )(h2d, y2d, comb, we1, be1.reshape(EXPERTS, 1, EMBED),
      we2, be2.reshape(EXPERTS, 1, EMBED))


def _head_body(h_ref, g_ref, bln_ref, w_ref, b_ref, o_ref):
    y = _ln(h_ref[...], g_ref[...], bln_ref[...])
    y3 = y.reshape(B, P, EMBED)
    pooled = y3[:, 0, :]
    for p in range(1, P):
        pooled = pooled + y3[:, p, :]
    pooled = pooled * (1.0 / P)
    o_ref[...] = jnp.dot(pooled, w_ref[...],
                         preferred_element_type=jnp.float32) + b_ref[...]


def _head(h2d, g, bln, w, b):
    return pl.pallas_call(
        _head_body,
        out_shape=jax.ShapeDtypeStruct((B, NUM_CLASSES), jnp.float32),
    )(h2d, g, bln, w, b)


def kernel(x, params):
    x2 = x.reshape(N * C * H, W)
    feats = _stem(x2, params['W_stem'], params['b_stem'].reshape(1, EMBED))

    h = feats
    for p in params['layers']:
        h3, y3, lg3 = _attn(h.reshape(B, P, EMBED),
                            p['ln1_g'].reshape(1, EMBED),
                            p['ln1_b'].reshape(1, EMBED),
                            p['Wqkv'], p['bqkv'].reshape(1, 3 * EMBED),
                            p['Wo'], p['bo'].reshape(1, EMBED),
                            p['ln2_g'].reshape(1, EMBED),
                            p['ln2_b'].reshape(1, EMBED), p['Wg'])
        comb = _comb_sc(lg3.reshape(N, EXPERTS))
        h = _moe(h3.reshape(N, EMBED), y3.reshape(N, EMBED), comb,
                 p['We1'], p['be1'], p['We2'], p['be2'])

    return _head(h, params['lnf_g'].reshape(1, EMBED),
                 params['lnf_b'].reshape(1, EMBED),
                 params['W_head'], params['b_head'].reshape(1, NUM_CLASSES))


# speedup vs baseline: 1.0332x; 1.0332x over previous
import functools

import jax
import jax.numpy as jnp
from jax.experimental import pallas as pl
from jax.experimental.pallas import tpu as pltpu
from jax.experimental.pallas import tpu_sc as plsc

EMBED = 512
EXPERTS = 16
TOPK = 4
HEADS = 16
DEPTH = 2
NUM_CLASSES = 1000
POOL = 14
B, P, C, H, W = 16, 16, 3, 56, 56
N = B * P
HD = EMBED // HEADS


def _ln(x, g, b):
    m = jnp.mean(x, axis=-1, keepdims=True)
    v = jnp.mean((x - m) ** 2, axis=-1, keepdims=True)
    return (x - m) / jnp.sqrt(v + 1e-6) * g + b


def _stem_body(x_ref, m_ref, w_ref, b_ref, o_ref):
    xb = x_ref[...]
    nc = xb.shape[0] // (C * H)
    x4 = xb.reshape(nc * C, POOL, 4, W)
    hs = x4[:, :, 0, :] + x4[:, :, 1, :] + x4[:, :, 2, :] + x4[:, :, 3, :]
    hs = hs.reshape(nc * C * POOL, W)
    s = jnp.dot(hs, m_ref[...], preferred_element_type=jnp.float32,
                precision=jax.lax.Precision.HIGHEST)
    s3 = s.reshape(nc, C * POOL, POOL)
    w3 = w_ref[...].reshape(C * POOL, POOL, EMBED)
    feats = b_ref[...]
    for k in range(C * POOL):
        feats = feats + jnp.dot(s3[:, k, :], w3[k],
                                preferred_element_type=jnp.float32)
    o_ref[...] = feats


def _stem(x2, w2, b):
    chunks = 8
    nc = N // chunks
    m = (jnp.arange(W)[:, None] // 4 == jnp.arange(POOL)[None, :])
    m = m.astype(jnp.float32) / 16.0
    return pl.pallas_call(
        _stem_body,
        grid=(chunks,),
        in_specs=[
            pl.BlockSpec((nc * C * H, W), lambda i: (i, 0)),
            pl.BlockSpec((W, POOL), lambda i: (0, 0)),
            pl.BlockSpec((C * POOL * POOL, EMBED), lambda i: (0, 0)),
            pl.BlockSpec((1, EMBED), lambda i: (0, 0)),
        ],
        out_specs=pl.BlockSpec((nc, EMBED), lambda i: (i, 0)),
        out_shape=jax.ShapeDtypeStruct((N, EMBED), jnp.float32),
    )(x2, m, w2, b)


def _attn_body(h_ref, g_ref, bln_ref, wqkv_ref, bqkv_ref, wo_ref, bo_ref,
               g2_ref, b2_ref, wg_ref, o_ref, y_ref, lg_ref):
    for bb in range(h_ref.shape[0]):
        _attn_one(bb, h_ref, g_ref, bln_ref, wqkv_ref, bqkv_ref, wo_ref,
                  bo_ref, g2_ref, b2_ref, wg_ref, o_ref, y_ref, lg_ref)


def _attn_one(bb, h_ref, g_ref, bln_ref, wqkv_ref, bqkv_ref, wo_ref, bo_ref,
              g2_ref, b2_ref, wg_ref, o_ref, y_ref, lg_ref):
    h = h_ref[bb]
    y = _ln(h, g_ref[...], bln_ref[...])
    qkv = jnp.dot(y, wqkv_ref[...], preferred_element_type=jnp.float32)
    qkv = qkv + bqkv_ref[...]

    qkvT = qkv.T
    q3 = qkvT[:EMBED].reshape(HEADS, HD, P)
    k3 = qkvT[EMBED:2 * EMBED].reshape(HEADS, HD, P)
    v3 = qkvT[2 * EMBED:].reshape(HEADS, HD, P)
    s = jax.lax.dot_general(q3, k3, (((1,), (1,)), ((0,), (0,))),
                            preferred_element_type=jnp.float32)
    s = s / jnp.sqrt(jnp.float32(HD))
    s = s - jnp.max(s, axis=-1, keepdims=True)
    ex = jnp.exp(s)
    att = ex / jnp.sum(ex, axis=-1, keepdims=True)
    o3 = jax.lax.dot_general(v3, att, (((2,), (2,)), ((0,), (0,))),
                             preferred_element_type=jnp.float32)
    wo = wo_ref[...]
    acc = h + bo_ref[...]
    for hh in range(HEADS):
        acc = acc + jax.lax.dot_general(
            o3[hh], wo[HD * hh:HD * (hh + 1), :], (((0,), (0,)), ((), ())),
            preferred_element_type=jnp.float32)
    o_ref[bb] = acc
    y2 = _ln(acc, g2_ref[...], b2_ref[...])
    y_ref[bb] = y2
    lg_ref[bb] = jnp.dot(y2, wg_ref[...], preferred_element_type=jnp.float32)


def _attn(h3, g, bln, wqkv, bqkv, wo, bo, g2, b2, wg):
    return pl.pallas_call(
        _attn_body,
        grid=(B // 4,),
        in_specs=[
            pl.BlockSpec((4, P, EMBED), lambda b: (b, 0, 0)),
            pl.BlockSpec((1, EMBED), lambda b: (0, 0)),
            pl.BlockSpec((1, EMBED), lambda b: (0, 0)),
            pl.BlockSpec((EMBED, 3 * EMBED), lambda b: (0, 0)),
            pl.BlockSpec((1, 3 * EMBED), lambda b: (0, 0)),
            pl.BlockSpec((EMBED, EMBED), lambda b: (0, 0)),
            pl.BlockSpec((1, EMBED), lambda b: (0, 0)),
            pl.BlockSpec((1, EMBED), lambda b: (0, 0)),
            pl.BlockSpec((1, EMBED), lambda b: (0, 0)),
            pl.BlockSpec((EMBED, EXPERTS), lambda b: (0, 0)),
        ],
        out_specs=[
            pl.BlockSpec((4, P, EMBED), lambda b: (b, 0, 0)),
            pl.BlockSpec((4, P, EMBED), lambda b: (b, 0, 0)),
            pl.BlockSpec((4, P, EXPERTS), lambda b: (b, 0, 0)),
        ],
        out_shape=[
            jax.ShapeDtypeStruct((B, P, EMBED), jnp.float32),
            jax.ShapeDtypeStruct((B, P, EMBED), jnp.float32),
            jax.ShapeDtypeStruct((B, P, EXPERTS), jnp.float32),
        ],
    )(h3, g, bln, wqkv, bqkv, wo, bo, g2, b2, wg)


def _comb_sc(logits):
    info = plsc.get_sparse_core_info()
    nw = info.num_cores * info.num_subcores
    rows = N // nw
    mesh = plsc.VectorSubcoreMesh(core_axis_name="c", subcore_axis_name="s")

    @functools.partial(
        pl.kernel, mesh=mesh,
        out_type=jax.ShapeDtypeStruct((N, EXPERTS), jnp.float32),
        scratch_types=[
            pltpu.VMEM((rows, EXPERTS), jnp.float32),
            pltpu.VMEM((rows, EXPERTS), jnp.float32),
        ],
        compiler_params=pltpu.CompilerParams(needs_layout_passes=False),
    )
    def k(l_hbm, out_hbm, l_v, o_v):
        wid = (jax.lax.axis_index("s") * info.num_cores
               + jax.lax.axis_index("c"))
        base = wid * rows
        pltpu.sync_copy(l_hbm.at[pl.ds(base, rows)], l_v)
        iot = jax.lax.iota(jnp.int32, 16)

        def rev(x):
            return jax.lax.rev(x, (0,))

        def bmax(x):
            return plsc.cummax(rev(plsc.cummax(x)))

        def bsum(x):
            return plsc.cumsum(x) + rev(plsc.cumsum(rev(x))) - x

        for r in range(rows):
            l = l_v[r]
            ex = jnp.exp(l - bmax(l))
            p = ex / bsum(ex)
            avail = iot >= 0
            sel = iot < 0
            for _ in range(TOPK):
                pm = jnp.where(avail, p, -1e30)
                mx = bmax(pm)
                cand = pm == mx
                idx = -bmax(jnp.where(cand, -iot, -EXPERTS))
                hit = iot == idx
                sel = sel | hit
                avail = avail & (~hit)
            cw = jnp.where(sel, p, 0.0)
            o_v[r] = cw / bsum(cw)
        pltpu.sync_copy(o_v, out_hbm.at[pl.ds(base, rows)])

    return k(logits)


def _moe_body(h_ref, y_ref, comb_ref, we1_ref, be1_ref, we2_ref,
              be2_ref, o_ref, acc_s):
    e = pl.program_id(0)
    y = y_ref[...]
    t = jnp.dot(y, we1_ref[0], preferred_element_type=jnp.float32)
    h1 = jax.nn.gelu(t + be1_ref[0])
    h2 = jnp.dot(h1, we2_ref[0], preferred_element_type=jnp.float32)
    h2 = h2 + be2_ref[0]
    lane = jax.lax.broadcasted_iota(jnp.int32, (N, EXPERTS), 1)
    ce = jnp.sum(comb_ref[...] * (lane == e).astype(jnp.float32), axis=1,
                 keepdims=True)
    contrib = (ce.astype(jnp.bfloat16).astype(jnp.float32) *
               h2.astype(jnp.bfloat16).astype(jnp.float32))

    @pl.when(e == 0)
    def _first():
        acc_s[...] = contrib

    @pl.when(e > 0)
    def _rest():
        acc_s[...] += contrib

    @pl.when(e == EXPERTS - 1)
    def _final():
        o_ref[...] = h_ref[...] + acc_s[...]


def _moe(h2d, y2d, comb, we1, be1, we2, be2):
    return pl.pallas_call(
        _moe_body,
        grid=(EXPERTS,),
        in_specs=[
            pl.BlockSpec((N, EMBED), lambda e: (0, 0)),
            pl.BlockSpec((N, EMBED), lambda e: (0, 0)),
            pl.BlockSpec((N, EXPERTS), lambda e: (0, 0)),
            pl.BlockSpec((1, EMBED, EMBED), lambda e: (e, 0, 0)),
            pl.BlockSpec((1, 1, EMBED), lambda e: (e, 0, 0)),
            pl.BlockSpec((1, EMBED, EMBED), lambda e: (e, 0, 0)),
            pl.BlockSpec((1, 1, EMBED), lambda e: (e, 0, 0)),
        ],
        out_specs=pl.BlockSpec((N, EMBED), lambda e: (0, 0)),
        out_shape=jax.ShapeDtypeStruct((N, EMBED), jnp.float32),
        scratch_shapes=[
            pltpu.VMEM((N, EMBED), jnp.float32),
        ],
    )(h2d, y2d, comb, we1, be1.reshape(EXPERTS, 1, EMBED),
      we2, be2.reshape(EXPERTS, 1, EMBED))


def _head_body(h_ref, g_ref, bln_ref, w_ref, b_ref, o_ref):
    y = _ln(h_ref[...], g_ref[...], bln_ref[...])
    y3 = y.reshape(B, P, EMBED)
    pooled = y3[:, 0, :]
    for p in range(1, P):
        pooled = pooled + y3[:, p, :]
    pooled = pooled * (1.0 / P)
    o_ref[...] = jnp.dot(pooled, w_ref[...],
                         preferred_element_type=jnp.float32) + b_ref[...]


def _head(h2d, g, bln, w, b):
    return pl.pallas_call(
        _head_body,
        out_shape=jax.ShapeDtypeStruct((B, NUM_CLASSES), jnp.float32),
    )(h2d, g, bln, w, b)


def kernel(x, params):
    x2 = x.reshape(N * C * H, W)
    feats = _stem(x2, params['W_stem'], params['b_stem'].reshape(1, EMBED))

    h = feats
    for p in params['layers']:
        h3, y3, lg3 = _attn(h.reshape(B, P, EMBED),
                            p['ln1_g'].reshape(1, EMBED),
                            p['ln1_b'].reshape(1, EMBED),
                            p['Wqkv'], p['bqkv'].reshape(1, 3 * EMBED),
                            p['Wo'], p['bo'].reshape(1, EMBED),
                            p['ln2_g'].reshape(1, EMBED),
                            p['ln2_b'].reshape(1, EMBED), p['Wg'])
        comb = _comb_sc(lg3.reshape(N, EXPERTS))
        h = _moe(h3.reshape(N, EMBED), y3.reshape(N, EMBED), comb,
                 p['We1'], p['be1'], p['We2'], p['be2'])

    return _head(h, params['lnf_g'].reshape(1, EMBED),
                 params['lnf_b'].reshape(1, EMBED),
                 params['W_head'], params['b_head'].reshape(1, NUM_CLASSES))


# speedup vs baseline: 1.0464x; 1.0128x over previous
import functools

import jax
import jax.numpy as jnp
from jax.experimental import pallas as pl
from jax.experimental.pallas import tpu as pltpu
from jax.experimental.pallas import tpu_sc as plsc

EMBED = 512
EXPERTS = 16
TOPK = 4
HEADS = 16
DEPTH = 2
NUM_CLASSES = 1000
POOL = 14
B, P, C, H, W = 16, 16, 3, 56, 56
N = B * P
HD = EMBED // HEADS


def _ln(x, g, b):
    m = jnp.mean(x, axis=-1, keepdims=True)
    v = jnp.mean((x - m) ** 2, axis=-1, keepdims=True)
    return (x - m) / jnp.sqrt(v + 1e-6) * g + b


def _stem_body(x_ref, m_ref, w_ref, b_ref, o_ref):
    xb = x_ref[...]
    nc = xb.shape[0] // (C * H)
    x4 = xb.reshape(nc * C, POOL, 4, W)
    hs = x4[:, :, 0, :] + x4[:, :, 1, :] + x4[:, :, 2, :] + x4[:, :, 3, :]
    hs = hs.reshape(nc * C * POOL, W)
    s = jnp.dot(hs, m_ref[...], preferred_element_type=jnp.float32,
                precision=jax.lax.Precision.HIGHEST)
    s3 = s.reshape(nc, C * POOL, POOL)
    w3 = w_ref[...].reshape(C * POOL, POOL, EMBED)
    feats = b_ref[...]
    for k in range(C * POOL):
        feats = feats + jnp.dot(s3[:, k, :], w3[k],
                                preferred_element_type=jnp.float32)
    o_ref[...] = feats


def _stem(x2, w2, b):
    chunks = 4
    nc = N // chunks
    m = (jnp.arange(W)[:, None] // 4 == jnp.arange(POOL)[None, :])
    m = m.astype(jnp.float32) / 16.0
    return pl.pallas_call(
        _stem_body,
        grid=(chunks,),
        in_specs=[
            pl.BlockSpec((nc * C * H, W), lambda i: (i, 0)),
            pl.BlockSpec((W, POOL), lambda i: (0, 0)),
            pl.BlockSpec((C * POOL * POOL, EMBED), lambda i: (0, 0)),
            pl.BlockSpec((1, EMBED), lambda i: (0, 0)),
        ],
        out_specs=pl.BlockSpec((nc, EMBED), lambda i: (i, 0)),
        out_shape=jax.ShapeDtypeStruct((N, EMBED), jnp.float32),
    )(x2, m, w2, b)


def _attn_body(h_ref, g_ref, bln_ref, wqkv_ref, bqkv_ref, wo_ref, bo_ref,
               g2_ref, b2_ref, wg_ref, o_ref, y_ref, lg_ref):
    for bb in range(h_ref.shape[0]):
        _attn_one(bb, h_ref, g_ref, bln_ref, wqkv_ref, bqkv_ref, wo_ref,
                  bo_ref, g2_ref, b2_ref, wg_ref, o_ref, y_ref, lg_ref)


def _attn_one(bb, h_ref, g_ref, bln_ref, wqkv_ref, bqkv_ref, wo_ref, bo_ref,
              g2_ref, b2_ref, wg_ref, o_ref, y_ref, lg_ref):
    h = h_ref[bb]
    y = _ln(h, g_ref[...], bln_ref[...])
    qkv = jnp.dot(y, wqkv_ref[...], preferred_element_type=jnp.float32)
    qkv = qkv + bqkv_ref[...]

    qkvT = qkv.T
    q3 = qkvT[:EMBED].reshape(HEADS, HD, P)
    k3 = qkvT[EMBED:2 * EMBED].reshape(HEADS, HD, P)
    v3 = qkvT[2 * EMBED:].reshape(HEADS, HD, P)
    s = jax.lax.dot_general(q3, k3, (((1,), (1,)), ((0,), (0,))),
                            preferred_element_type=jnp.float32)
    s = s / jnp.sqrt(jnp.float32(HD))
    s = s - jnp.max(s, axis=-1, keepdims=True)
    ex = jnp.exp(s)
    att = ex / jnp.sum(ex, axis=-1, keepdims=True)
    o3 = jax.lax.dot_general(v3, att, (((2,), (2,)), ((0,), (0,))),
                             preferred_element_type=jnp.float32)
    wo = wo_ref[...]
    acc = h + bo_ref[...]
    for hh in range(HEADS):
        acc = acc + jax.lax.dot_general(
            o3[hh], wo[HD * hh:HD * (hh + 1), :], (((0,), (0,)), ((), ())),
            preferred_element_type=jnp.float32)
    o_ref[bb] = acc
    y2 = _ln(acc, g2_ref[...], b2_ref[...])
    y_ref[bb] = y2
    lg_ref[bb] = jnp.dot(y2, wg_ref[...], preferred_element_type=jnp.float32)


def _attn(h3, g, bln, wqkv, bqkv, wo, bo, g2, b2, wg):
    return pl.pallas_call(
        _attn_body,
        grid=(B // 4,),
        in_specs=[
            pl.BlockSpec((4, P, EMBED), lambda b: (b, 0, 0)),
            pl.BlockSpec((1, EMBED), lambda b: (0, 0)),
            pl.BlockSpec((1, EMBED), lambda b: (0, 0)),
            pl.BlockSpec((EMBED, 3 * EMBED), lambda b: (0, 0)),
            pl.BlockSpec((1, 3 * EMBED), lambda b: (0, 0)),
            pl.BlockSpec((EMBED, EMBED), lambda b: (0, 0)),
            pl.BlockSpec((1, EMBED), lambda b: (0, 0)),
            pl.BlockSpec((1, EMBED), lambda b: (0, 0)),
            pl.BlockSpec((1, EMBED), lambda b: (0, 0)),
            pl.BlockSpec((EMBED, EXPERTS), lambda b: (0, 0)),
        ],
        out_specs=[
            pl.BlockSpec((4, P, EMBED), lambda b: (b, 0, 0)),
            pl.BlockSpec((4, P, EMBED), lambda b: (b, 0, 0)),
            pl.BlockSpec((4, P, EXPERTS), lambda b: (b, 0, 0)),
        ],
        out_shape=[
            jax.ShapeDtypeStruct((B, P, EMBED), jnp.float32),
            jax.ShapeDtypeStruct((B, P, EMBED), jnp.float32),
            jax.ShapeDtypeStruct((B, P, EXPERTS), jnp.float32),
        ],
    )(h3, g, bln, wqkv, bqkv, wo, bo, g2, b2, wg)


def _comb_sc(logits):
    info = plsc.get_sparse_core_info()
    nw = info.num_cores * info.num_subcores
    rows = N // nw
    mesh = plsc.VectorSubcoreMesh(core_axis_name="c", subcore_axis_name="s")

    @functools.partial(
        pl.kernel, mesh=mesh,
        out_type=jax.ShapeDtypeStruct((N, EXPERTS), jnp.float32),
        scratch_types=[
            pltpu.VMEM((rows, EXPERTS), jnp.float32),
            pltpu.VMEM((rows, EXPERTS), jnp.float32),
        ],
        compiler_params=pltpu.CompilerParams(needs_layout_passes=False),
    )
    def k(l_hbm, out_hbm, l_v, o_v):
        wid = (jax.lax.axis_index("s") * info.num_cores
               + jax.lax.axis_index("c"))
        base = wid * rows
        pltpu.sync_copy(l_hbm.at[pl.ds(base, rows)], l_v)
        iot = jax.lax.iota(jnp.int32, 16)

        def rev(x):
            return jax.lax.rev(x, (0,))

        def bmax(x):
            return plsc.cummax(rev(plsc.cummax(x)))

        def bsum(x):
            return plsc.cumsum(x) + rev(plsc.cumsum(rev(x))) - x

        for r in range(rows):
            l = l_v[r]
            ex = jnp.exp(l - bmax(l))
            p = ex / bsum(ex)
            avail = iot >= 0
            sel = iot < 0
            for _ in range(TOPK):
                pm = jnp.where(avail, p, -1e30)
                mx = bmax(pm)
                cand = pm == mx
                idx = -bmax(jnp.where(cand, -iot, -EXPERTS))
                hit = iot == idx
                sel = sel | hit
                avail = avail & (~hit)
            cw = jnp.where(sel, p, 0.0)
            o_v[r] = cw / bsum(cw)
        pltpu.sync_copy(o_v, out_hbm.at[pl.ds(base, rows)])

    return k(logits)


def _moe_body(h_ref, y_ref, comb_ref, we1_ref, be1_ref, we2_ref,
              be2_ref, o_ref, acc_s):
    e = pl.program_id(0)
    y = y_ref[...]
    t = jnp.dot(y, we1_ref[0], preferred_element_type=jnp.float32)
    h1 = jax.nn.gelu(t + be1_ref[0])
    h2 = jnp.dot(h1, we2_ref[0], preferred_element_type=jnp.float32)
    h2 = h2 + be2_ref[0]
    lane = jax.lax.broadcasted_iota(jnp.int32, (N, EXPERTS), 1)
    ce = jnp.sum(comb_ref[...] * (lane == e).astype(jnp.float32), axis=1,
                 keepdims=True)
    contrib = (ce.astype(jnp.bfloat16).astype(jnp.float32) *
               h2.astype(jnp.bfloat16).astype(jnp.float32))

    @pl.when(e == 0)
    def _first():
        acc_s[...] = contrib

    @pl.when(e > 0)
    def _rest():
        acc_s[...] += contrib

    @pl.when(e == EXPERTS - 1)
    def _final():
        o_ref[...] = h_ref[...] + acc_s[...]


def _moe(h2d, y2d, comb, we1, be1, we2, be2):
    return pl.pallas_call(
        _moe_body,
        grid=(EXPERTS,),
        in_specs=[
            pl.BlockSpec((N, EMBED), lambda e: (0, 0)),
            pl.BlockSpec((N, EMBED), lambda e: (0, 0)),
            pl.BlockSpec((N, EXPERTS), lambda e: (0, 0)),
            pl.BlockSpec((1, EMBED, EMBED), lambda e: (e, 0, 0)),
            pl.BlockSpec((1, 1, EMBED), lambda e: (e, 0, 0)),
            pl.BlockSpec((1, EMBED, EMBED), lambda e: (e, 0, 0)),
            pl.BlockSpec((1, 1, EMBED), lambda e: (e, 0, 0)),
        ],
        out_specs=pl.BlockSpec((N, EMBED), lambda e: (0, 0)),
        out_shape=jax.ShapeDtypeStruct((N, EMBED), jnp.float32),
        scratch_shapes=[
            pltpu.VMEM((N, EMBED), jnp.float32),
        ],
    )(h2d, y2d, comb, we1, be1.reshape(EXPERTS, 1, EMBED),
      we2, be2.reshape(EXPERTS, 1, EMBED))


def _moe_head_body(h_ref, y_ref, comb_ref, we1_ref, be1_ref, we2_ref,
                   be2_ref, gf_ref, bf_ref, wh_ref, bh_ref, o_ref, acc_s):
    e = pl.program_id(0)
    y = y_ref[...]
    t = jnp.dot(y, we1_ref[0], preferred_element_type=jnp.float32)
    h1 = jax.nn.gelu(t + be1_ref[0])
    h2 = jnp.dot(h1, we2_ref[0], preferred_element_type=jnp.float32)
    h2 = h2 + be2_ref[0]
    lane = jax.lax.broadcasted_iota(jnp.int32, (N, EXPERTS), 1)
    ce = jnp.sum(comb_ref[...] * (lane == e).astype(jnp.float32), axis=1,
                 keepdims=True)
    contrib = (ce.astype(jnp.bfloat16).astype(jnp.float32) *
               h2.astype(jnp.bfloat16).astype(jnp.float32))

    @pl.when(e == 0)
    def _first():
        acc_s[...] = contrib

    @pl.when(e > 0)
    def _rest():
        acc_s[...] += contrib

    @pl.when(e == EXPERTS - 1)
    def _final():
        hfin = h_ref[...] + acc_s[...]
        yf = _ln(hfin, gf_ref[...], bf_ref[...])
        y3 = yf.reshape(B, P, EMBED)
        pooled = y3[:, 0, :]
        for p in range(1, P):
            pooled = pooled + y3[:, p, :]
        pooled = pooled * (1.0 / P)
        o_ref[...] = jnp.dot(pooled, wh_ref[...],
                             preferred_element_type=jnp.float32) + bh_ref[...]


def _moe_head(h2d, y2d, comb, we1, be1, we2, be2, gf, bf, wh, bh):
    return pl.pallas_call(
        _moe_head_body,
        grid=(EXPERTS,),
        in_specs=[
            pl.BlockSpec((N, EMBED), lambda e: (0, 0)),
            pl.BlockSpec((N, EMBED), lambda e: (0, 0)),
            pl.BlockSpec((N, EXPERTS), lambda e: (0, 0)),
            pl.BlockSpec((1, EMBED, EMBED), lambda e: (e, 0, 0)),
            pl.BlockSpec((1, 1, EMBED), lambda e: (e, 0, 0)),
            pl.BlockSpec((1, EMBED, EMBED), lambda e: (e, 0, 0)),
            pl.BlockSpec((1, 1, EMBED), lambda e: (e, 0, 0)),
            pl.BlockSpec((1, EMBED), lambda e: (0, 0)),
            pl.BlockSpec((1, EMBED), lambda e: (0, 0)),
            pl.BlockSpec((EMBED, NUM_CLASSES), lambda e: (0, 0)),
            pl.BlockSpec((1, NUM_CLASSES), lambda e: (0, 0)),
        ],
        out_specs=pl.BlockSpec((B, NUM_CLASSES), lambda e: (0, 0)),
        out_shape=jax.ShapeDtypeStruct((B, NUM_CLASSES), jnp.float32),
        scratch_shapes=[
            pltpu.VMEM((N, EMBED), jnp.float32),
        ],
    )(h2d, y2d, comb, we1, be1.reshape(EXPERTS, 1, EMBED),
      we2, be2.reshape(EXPERTS, 1, EMBED), gf, bf, wh, bh)


def _head_body(h_ref, g_ref, bln_ref, w_ref, b_ref, o_ref):
    y = _ln(h_ref[...], g_ref[...], bln_ref[...])
    y3 = y.reshape(B, P, EMBED)
    pooled = y3[:, 0, :]
    for p in range(1, P):
        pooled = pooled + y3[:, p, :]
    pooled = pooled * (1.0 / P)
    o_ref[...] = jnp.dot(pooled, w_ref[...],
                         preferred_element_type=jnp.float32) + b_ref[...]


def _head(h2d, g, bln, w, b):
    return pl.pallas_call(
        _head_body,
        out_shape=jax.ShapeDtypeStruct((B, NUM_CLASSES), jnp.float32),
    )(h2d, g, bln, w, b)


def kernel(x, params):
    x2 = x.reshape(N * C * H, W)
    feats = _stem(x2, params['W_stem'], params['b_stem'].reshape(1, EMBED))

    h = feats
    for i, p in enumerate(params['layers']):
        h3, y3, lg3 = _attn(h.reshape(B, P, EMBED),
                            p['ln1_g'].reshape(1, EMBED),
                            p['ln1_b'].reshape(1, EMBED),
                            p['Wqkv'], p['bqkv'].reshape(1, 3 * EMBED),
                            p['Wo'], p['bo'].reshape(1, EMBED),
                            p['ln2_g'].reshape(1, EMBED),
                            p['ln2_b'].reshape(1, EMBED), p['Wg'])
        comb = _comb_sc(lg3.reshape(N, EXPERTS))
        if i < len(params['layers']) - 1:
            h = _moe(h3.reshape(N, EMBED), y3.reshape(N, EMBED), comb,
                     p['We1'], p['be1'], p['We2'], p['be2'])
        else:
            return _moe_head(h3.reshape(N, EMBED), y3.reshape(N, EMBED),
                             comb, p['We1'], p['be1'], p['We2'], p['be2'],
                             params['lnf_g'].reshape(1, EMBED),
                             params['lnf_b'].reshape(1, EMBED),
                             params['W_head'],
                             params['b_head'].reshape(1, NUM_CLASSES))
